# Initial kernel scaffold; baseline (speedup 1.0000x reference)
import jax
import jax.numpy as jnp
from jax.experimental import pallas as pl


def kernel(pois_embs, edge_index, edge_weight):
    raise NotImplementedError("write your pallas kernel here")


# speedup vs baseline: 3.2697x; 3.2697x over previous
import jax
import jax.numpy as jnp
from jax import lax
from jax.experimental import pallas as pl
from jax.experimental.pallas import tpu as pltpu, tpu_sc as plsc

L_NODES = 10000
L_PAD = 10240
D_FEAT = 128
E_EDGES = 320000
N_LAYERS = 3

NC = 2
NS = 16
CH = 128
NB = 40
GROUPS = 2
CHUNKS = NB * GROUPS
EP = CHUNKS * CH * NS * NC
ROWS_PER_TILE = L_PAD // NS
RCH = 128
RCHUNKS = ROWS_PER_TILE // RCH


def _body(x_hbm, src_hbm, dst_hbm, w_hbm, out_hbm, agg_sh, srcb, dstb, wb, gbuf):
    c = lax.axis_index("c")
    s = lax.axis_index("s")
    row0 = pl.multiple_of(s * ROWS_PER_TILE, ROWS_PER_TILE)

    def _zero_row(r, _):
        for k in range(D_FEAT // 16):
            gbuf[r, pl.ds(k * 16, 16)] = jnp.zeros((16,), jnp.float32)
        return 0
    lax.fori_loop(0, CH, _zero_row, 0)
    for r in range(RCHUNKS):
        pltpu.sync_copy(gbuf, agg_sh.at[pl.ds(row0 + r * RCH, RCH)])
    plsc.subcore_barrier()

    def _edge_chunk(j, _):
        pltpu.sync_copy(x_hbm.at[srcb.at[j, 0]], gbuf)

        def _scale(g, _):
            w16 = wb[j, 0, pl.ds(g * 16, 16)]
            for i in range(16):
                e = g * 16 + i
                wi = w16[i]
                for k in range(D_FEAT // 16):
                    sl = pl.ds(k * 16, 16)
                    gbuf[e, sl] = gbuf[e, sl] * wi
            return 0
        lax.fori_loop(0, CH // 16, _scale, 0)

        pltpu.sync_copy(gbuf, agg_sh.at[dstb.at[j, 0]], add=True)
        return 0

    def _edge_group(g, _):
        base = pl.multiple_of(g * NB, NB)
        pltpu.sync_copy(src_hbm.at[c, s, pl.ds(base, NB)], srcb)
        pltpu.sync_copy(dst_hbm.at[c, s, pl.ds(base, NB)], dstb)
        pltpu.sync_copy(w_hbm.at[c, s, pl.ds(base, NB)], wb)
        lax.fori_loop(0, NB, _edge_chunk, 0)
        return 0

    lax.fori_loop(0, GROUPS, _edge_group, 0)
    plsc.subcore_barrier()

    for r in range(RCHUNKS):
        base = row0 + r * RCH
        pltpu.sync_copy(agg_sh.at[pl.ds(base, RCH)], gbuf)
        pltpu.sync_copy(gbuf, out_hbm.at[c, pl.ds(base, RCH)])


@jax.jit
def _run(xk, srcp, dstp, wp):
    mesh = plsc.VectorSubcoreMesh(core_axis_name="c", subcore_axis_name="s")
    f = pl.kernel(
        _body,
        mesh=mesh,
        out_type=jax.ShapeDtypeStruct((NC, L_PAD, D_FEAT), jnp.float32),
        scratch_types=[
            pltpu.VMEM_SHARED((L_PAD, D_FEAT), jnp.float32),
            pltpu.VMEM((NB, 1, CH), jnp.int32),
            pltpu.VMEM((NB, 1, CH), jnp.int32),
            pltpu.VMEM((NB, 1, CH), jnp.float32),
            pltpu.VMEM((CH, D_FEAT), jnp.float32),
        ],
    )
    return f(xk, srcp, dstp, wp)


def kernel(pois_embs, edge_index, edge_weight):
    src = edge_index[0]
    dst = edge_index[1]
    pad = EP - E_EDGES
    srcp = jnp.concatenate([src, jnp.zeros((pad,), jnp.int32)]).reshape(NC, NS, CHUNKS, 1, CH)
    dstp = jnp.concatenate([dst, jnp.zeros((pad,), jnp.int32)]).reshape(NC, NS, CHUNKS, 1, CH)
    wp = jnp.concatenate([edge_weight, jnp.zeros((pad,), jnp.float32)]).reshape(NC, NS, CHUNKS, 1, CH)
    x = jnp.concatenate(
        [pois_embs, jnp.zeros((L_PAD - L_NODES, D_FEAT), jnp.float32)])
    acc = x
    for _ in range(N_LAYERS):
        partials = _run(x, srcp, dstp, wp)
        x = partials[0] + partials[1] + x
        acc = acc + x
    return (acc * 0.25)[:L_NODES]
